# Initial kernel scaffold; baseline (speedup 1.0000x reference)
import jax
import jax.numpy as jnp
from jax.experimental import pallas as pl


def kernel(embeddings, instance_labels):
    raise NotImplementedError("write your pallas kernel here")


# speedup vs baseline: 34.1391x; 34.1391x over previous
import jax
import jax.numpy as jnp
from jax.experimental import pallas as pl
from jax.experimental.pallas import tpu as pltpu

_B, _E, _HW = 8, 32, 224 * 224
_K = 16
_DELTA_VAR, _DELTA_DIST = 0.5, 1.5
_ALPHA, _BETA, _GAMMA = 1.0, 1.0, 0.001
_PB = 7168
_NB = _HW // _PB

_HI = jax.lax.Precision.HIGHEST


def _loss_kernel(lab_ref, e_ref, out_ref, sums_ref, counts_ref, acc_ref):
    p = pl.program_id(1)
    j = pl.program_id(2)

    e = e_ref[0]
    lab = lab_ref[0]
    kio = jax.lax.broadcasted_iota(jnp.int32, (_K, _PB), 0)
    oh = (kio == lab).astype(jnp.float32)

    @pl.when(jnp.logical_and(p == 0, j == 0))
    def _init():
        sums_ref[...] = jnp.zeros_like(sums_ref)
        counts_ref[...] = jnp.zeros_like(counts_ref)
        acc_ref[0] = 0.0

    @pl.when(p == 0)
    def _pass_sums():
        s = jax.lax.dot_general(oh, e, (((1,), (1,)), ((), ())), precision=_HI)
        sums_ref[...] += s
        cp = oh[:, 0:128]
        for i in range(1, _PB // 128):
            cp = cp + oh[:, i * 128:(i + 1) * 128]
        counts_ref[...] += cp

    @pl.when(p == 1)
    def _pass_var():
        cnt = jnp.sum(counts_ref[...], axis=1, keepdims=True)
        means = sums_ref[...] / jnp.maximum(cnt, 1.0)
        mm2 = jnp.sum(means * means, axis=1, keepdims=True)
        dot = jax.lax.dot_general(means, e, (((1,), (0,)), ((), ())),
                                  precision=_HI)
        sel = jnp.sum(oh * (mm2 - 2.0 * dot), axis=0, keepdims=True)
        ee = jnp.sum(e * e, axis=0, keepdims=True)
        d2 = ee + sel
        dist = jnp.sqrt(jnp.maximum(d2, 0.0))
        valid = (lab > 0).astype(jnp.float32)
        hinge = jnp.maximum(dist - _DELTA_VAR, 0.0) * valid
        acc_ref[0] += jnp.sum(hinge)

    @pl.when(jnp.logical_and(p == 1, j == _NB - 1))
    def _finalize():
        cnt = jnp.sum(counts_ref[...], axis=1, keepdims=True)
        ids = jax.lax.broadcasted_iota(jnp.int32, (_K, 1), 0)
        presf = jnp.logical_and(cnt > 0.0, ids > 0).astype(jnp.float32)
        n_inst = jnp.sum(presf)
        means = sums_ref[...] / jnp.maximum(cnt, 1.0)
        mm2 = jnp.sum(means * means, axis=1, keepdims=True)
        var_loss = acc_ref[0] / jnp.maximum(n_inst, 1.0)
        g = jax.lax.dot_general(means, means, (((1,), (1,)), ((), ())),
                                precision=_HI)
        p2 = jax.lax.dot_general(presf, presf, (((1,), (1,)), ((), ())),
                                 precision=_HI)
        ir = jax.lax.broadcasted_iota(jnp.int32, (_K, _K), 0)
        ic = jax.lax.broadcasted_iota(jnp.int32, (_K, _K), 1)
        eye = (ir == ic).astype(jnp.float32)
        ge = g * eye
        mm2c = jnp.sum(ge, axis=1, keepdims=True)
        mm2r = jnp.sum(ge, axis=0, keepdims=True)
        pd2 = mm2c + mm2r - 2.0 * g
        pd = jnp.sqrt(jnp.maximum(pd2, 0.0))
        hingep = jnp.maximum(2.0 * _DELTA_DIST - pd, 0.0)
        tri = (ir < ic).astype(jnp.float32)
        pairsum = jnp.sum(hingep * p2 * tri)
        n_pairs = n_inst * (n_inst - 1.0) * 0.5
        dist_loss = jnp.where(n_inst > 1.0,
                              pairsum / jnp.maximum(n_pairs, 1.0), 0.0)
        mnorm = jnp.sqrt(jnp.maximum(mm2, 0.0))
        reg_loss = jnp.sum(presf * mnorm) / jnp.maximum(n_inst, 1.0)
        total = (_ALPHA * var_loss + _BETA * dist_loss + _GAMMA * reg_loss)
        out_ref[...] = jnp.broadcast_to(total, (1, 1, 1))


def kernel(embeddings, instance_labels):
    e3 = embeddings.reshape(_B, _E, _HW)
    lab3 = instance_labels.reshape(_B * _NB, 1, _PB)
    per_batch = pl.pallas_call(
        _loss_kernel,
        grid=(_B, 2, _NB),
        in_specs=[
            pl.BlockSpec((1, 1, _PB), lambda b, p, j: (b * _NB + j, 0, 0)),
            pl.BlockSpec((1, _E, _PB), lambda b, p, j: (b, 0, j)),
        ],
        out_specs=pl.BlockSpec((1, 1, 1), lambda b, p, j: (b, 0, 0)),
        out_shape=jax.ShapeDtypeStruct((_B, 1, 1), jnp.float32),
        scratch_shapes=[
            pltpu.VMEM((_K, _E), jnp.float32),
            pltpu.VMEM((_K, 128), jnp.float32),
            pltpu.SMEM((1,), jnp.float32),
        ],
    )(lab3, e3)
    return jnp.sum(per_batch) / _B


# speedup vs baseline: 36.4415x; 1.0674x over previous
import jax
import jax.numpy as jnp
from jax.experimental import pallas as pl
from jax.experimental.pallas import tpu as pltpu

_B, _E, _HW = 8, 32, 224 * 224
_K = 16
_DELTA_VAR, _DELTA_DIST = 0.5, 1.5
_ALPHA, _BETA, _GAMMA = 1.0, 1.0, 0.001
_PB = 7168
_NB = _HW // _PB

_HI = jax.lax.Precision.HIGHEST


def _loss_kernel(lab_ref, e_ref, out_ref, sums_ref, counts_ref, acc_ref):
    p = pl.program_id(1)
    j = pl.program_id(2)

    e = e_ref[0]
    lab = lab_ref[0]
    kio = jax.lax.broadcasted_iota(jnp.int32, (_K, _PB), 0)
    oh = (kio == lab).astype(jnp.float32)

    @pl.when(jnp.logical_and(p == 0, j == 0))
    def _init():
        sums_ref[...] = jnp.zeros_like(sums_ref)
        counts_ref[...] = jnp.zeros_like(counts_ref)
        acc_ref[0] = 0.0

    @pl.when(p == 0)
    def _pass_sums():
        s = jax.lax.dot_general(oh, e, (((1,), (1,)), ((), ())), precision=_HI)
        sums_ref[...] += s
        cp = oh[:, 0:128]
        for i in range(1, _PB // 128):
            cp = cp + oh[:, i * 128:(i + 1) * 128]
        counts_ref[...] += cp

    @pl.when(p == 1)
    def _pass_var():
        cnt = jnp.sum(counts_ref[...], axis=1, keepdims=True)
        means = sums_ref[...] / jnp.maximum(cnt, 1.0)
        mm2 = jnp.sum(means * means, axis=1, keepdims=True)
        dot2 = jax.lax.dot_general(2.0 * means, e, (((1,), (0,)), ((), ())),
                                   precision=jax.lax.Precision.DEFAULT)
        sel = jnp.sum(oh * (mm2 - dot2), axis=0, keepdims=True)
        ee = jnp.sum(e * e, axis=0, keepdims=True)
        d2 = ee + sel
        dist = jnp.sqrt(jnp.maximum(d2, 0.0))
        valid = (lab > 0).astype(jnp.float32)
        hinge = jnp.maximum(dist - _DELTA_VAR, 0.0) * valid
        acc_ref[0] += jnp.sum(hinge)

    @pl.when(jnp.logical_and(p == 1, j == _NB - 1))
    def _finalize():
        cnt = jnp.sum(counts_ref[...], axis=1, keepdims=True)
        ids = jax.lax.broadcasted_iota(jnp.int32, (_K, 1), 0)
        presf = jnp.logical_and(cnt > 0.0, ids > 0).astype(jnp.float32)
        n_inst = jnp.sum(presf)
        means = sums_ref[...] / jnp.maximum(cnt, 1.0)
        mm2 = jnp.sum(means * means, axis=1, keepdims=True)
        var_loss = acc_ref[0] / jnp.maximum(n_inst, 1.0)
        g = jax.lax.dot_general(means, means, (((1,), (1,)), ((), ())),
                                precision=_HI)
        p2 = jax.lax.dot_general(presf, presf, (((1,), (1,)), ((), ())),
                                 precision=_HI)
        ir = jax.lax.broadcasted_iota(jnp.int32, (_K, _K), 0)
        ic = jax.lax.broadcasted_iota(jnp.int32, (_K, _K), 1)
        eye = (ir == ic).astype(jnp.float32)
        ge = g * eye
        mm2c = jnp.sum(ge, axis=1, keepdims=True)
        mm2r = jnp.sum(ge, axis=0, keepdims=True)
        pd2 = mm2c + mm2r - 2.0 * g
        pd = jnp.sqrt(jnp.maximum(pd2, 0.0))
        hingep = jnp.maximum(2.0 * _DELTA_DIST - pd, 0.0)
        tri = (ir < ic).astype(jnp.float32)
        pairsum = jnp.sum(hingep * p2 * tri)
        n_pairs = n_inst * (n_inst - 1.0) * 0.5
        dist_loss = jnp.where(n_inst > 1.0,
                              pairsum / jnp.maximum(n_pairs, 1.0), 0.0)
        mnorm = jnp.sqrt(jnp.maximum(mm2, 0.0))
        reg_loss = jnp.sum(presf * mnorm) / jnp.maximum(n_inst, 1.0)
        total = (_ALPHA * var_loss + _BETA * dist_loss + _GAMMA * reg_loss)
        out_ref[...] = jnp.broadcast_to(total, (1, 1, 1))


def kernel(embeddings, instance_labels):
    e3 = embeddings.reshape(_B, _E, _HW)
    lab3 = instance_labels.reshape(_B * _NB, 1, _PB)
    per_batch = pl.pallas_call(
        _loss_kernel,
        grid=(_B, 2, _NB),
        in_specs=[
            pl.BlockSpec((1, 1, _PB), lambda b, p, j: (b * _NB + j, 0, 0)),
            pl.BlockSpec((1, _E, _PB), lambda b, p, j: (b, 0, j)),
        ],
        out_specs=pl.BlockSpec((1, 1, 1), lambda b, p, j: (b, 0, 0)),
        out_shape=jax.ShapeDtypeStruct((_B, 1, 1), jnp.float32),
        scratch_shapes=[
            pltpu.VMEM((_K, _E), jnp.float32),
            pltpu.VMEM((_K, 128), jnp.float32),
            pltpu.SMEM((1,), jnp.float32),
        ],
    )(lab3, e3)
    return jnp.sum(per_batch) / _B
